# Initial kernel scaffold; baseline (speedup 1.0000x reference)
import jax
import jax.numpy as jnp
from jax.experimental import pallas as pl


def kernel(x, edge_index, reachability_edge_index, W1, b1, W2, b2, L1W, L1b, L2W, L2b):
    raise NotImplementedError("write your pallas kernel here")


# speedup vs baseline: 4.2473x; 4.2473x over previous
import functools

import jax
import jax.numpy as jnp
from jax import lax
from jax.experimental import pallas as pl
from jax.experimental.pallas import tpu as pltpu
from jax.experimental.pallas import tpu_sc as plsc

N_NODES = 10000
D_FEAT = 128
HIDDEN = 16
N_EDGES = 320000
N_REACH = 500000

NC = 2
NS = 16
NW = NC * NS
L = 16

NPAD = 10240
ROWS_PER_TILE = NPAD // NS

G = 128
NBE = 79
E_TILE = NBE * G
E_PAD = NW * E_TILE

NBR = 125
R_TILE = NBR * G
R_PAD = NW * R_TILE

_sc_mesh = plsc.VectorSubcoreMesh(core_axis_name="c", subcore_axis_name="s")


@functools.partial(
    pl.kernel,
    out_type=jax.ShapeDtypeStruct((NC, NPAD, HIDDEN), jnp.float32),
    mesh=_sc_mesh,
    compiler_params=pltpu.CompilerParams(use_tc_tiling_on_sc=False, needs_layout_passes=False),
    scratch_types=[
        pltpu.VMEM((NBE, G), jnp.int32),
        pltpu.VMEM((G, HIDDEN), jnp.float32),
        pltpu.VMEM_SHARED((NPAD, HIDDEN), jnp.float32),
    ],
)
def _k_deg(dst_hbm, zeros_hbm, ones_hbm, out_hbm, idx_v, ones_v, acc_sh):
    c = lax.axis_index("c")
    s = lax.axis_index("s")
    wid = c * NS + s
    sl = pl.ds(s * ROWS_PER_TILE, ROWS_PER_TILE)
    pltpu.sync_copy(zeros_hbm.at[sl], acc_sh.at[sl])
    pltpu.sync_copy(ones_hbm, ones_v)
    pltpu.sync_copy(dst_hbm.at[wid], idx_v)
    plsc.subcore_barrier()

    def body(b, carry):
        pltpu.sync_copy(ones_v, acc_sh.at[idx_v.at[b]], add=True)
        return carry

    lax.fori_loop(0, NBE, body, 0)
    plsc.subcore_barrier()
    pltpu.sync_copy(acc_sh.at[sl], out_hbm.at[c, sl])


@functools.partial(
    pl.kernel,
    out_type=jax.ShapeDtypeStruct((NC, NPAD, HIDDEN), jnp.float32),
    mesh=_sc_mesh,
    compiler_params=pltpu.CompilerParams(use_tc_tiling_on_sc=False, needs_layout_passes=False),
    scratch_types=[
        pltpu.VMEM((NBE, G), jnp.int32),
        pltpu.VMEM((NBE, G), jnp.int32),
        pltpu.VMEM((G, HIDDEN), jnp.float32),
        pltpu.VMEM_SHARED((NPAD, HIDDEN), jnp.float32),
        pltpu.SemaphoreType.DMA,
    ],
)
def _k_scatter(u_hbm, src_hbm, dst_hbm, zeros_hbm, out_hbm,
               si_v, di_v, rows_v, acc_sh, sem):
    c = lax.axis_index("c")
    s = lax.axis_index("s")
    wid = c * NS + s
    sl = pl.ds(s * ROWS_PER_TILE, ROWS_PER_TILE)
    pltpu.sync_copy(zeros_hbm.at[sl], acc_sh.at[sl])
    pltpu.sync_copy(src_hbm.at[wid], si_v)
    pltpu.sync_copy(dst_hbm.at[wid], di_v)
    plsc.subcore_barrier()

    def body(b, carry):
        pltpu.async_copy(u_hbm.at[si_v.at[b]], rows_v, sem).wait()
        pltpu.sync_copy(rows_v, acc_sh.at[di_v.at[b]], add=True)
        return carry

    lax.fori_loop(0, NBE, body, 0)
    plsc.subcore_barrier()
    pltpu.sync_copy(acc_sh.at[sl], out_hbm.at[c, sl])


@functools.partial(
    pl.kernel,
    out_type=jax.ShapeDtypeStruct((NW, R_TILE), jnp.float32),
    mesh=_sc_mesh,
    compiler_params=pltpu.CompilerParams(use_tc_tiling_on_sc=False, needs_layout_passes=False),
    scratch_types=[
        pltpu.VMEM((NBR, G), jnp.int32),
        pltpu.VMEM((NBR, G), jnp.int32),
        pltpu.VMEM((G, D_FEAT), jnp.float32),
        pltpu.VMEM((D_FEAT,), jnp.float32),
        pltpu.VMEM((L,), jnp.float32),
        pltpu.VMEM((R_TILE,), jnp.float32),
        pltpu.SemaphoreType.DMA,
        pltpu.SemaphoreType.DMA,
    ],
)
def _k_decode(a_hbm, b_hbm, s_hbm, t_hbm, w_hbm, l2b_hbm, out_hbm,
              si_v, ti_v, rows_v, w_v, l2b_v, out_v, sem_a, sem_b):
    c = lax.axis_index("c")
    s = lax.axis_index("s")
    wid = c * NS + s
    pltpu.sync_copy(s_hbm.at[wid], si_v)
    pltpu.sync_copy(t_hbm.at[wid], ti_v)
    pltpu.sync_copy(w_hbm, w_v)
    pltpu.sync_copy(l2b_hbm, l2b_v)
    bias = l2b_v[...]
    row_ids = [lax.iota(jnp.int32, L) + (L * g) for g in range(G // L)]
    zero16 = jnp.zeros((L,), jnp.float32)

    def body(b, carry):
        pltpu.async_copy(a_hbm.at[si_v.at[b]], rows_v, sem_a).wait()
        pltpu.async_copy(b_hbm.at[ti_v.at[b]], rows_v, sem_b, add=True).wait()

        def jbody(j, accs):
            col = jnp.zeros((L,), jnp.int32) + j
            wj = plsc.load_gather(w_v, [col])
            out = []
            for g in range(G // L):
                v = plsc.load_gather(rows_v, [row_ids[g], col])
                out.append(accs[g] + jnp.maximum(v, 0.0) * wj)
            return tuple(out)

        accs = lax.fori_loop(0, D_FEAT, jbody,
                             tuple(zero16 for _ in range(G // L)))
        for g in range(G // L):
            z = accs[g] + bias
            out_v[pl.ds(b * G + L * g, L)] = 1.0 / (1.0 + jnp.exp(-z))
        return carry

    lax.fori_loop(0, NBR, body, 0)
    pltpu.sync_copy(out_v, out_hbm.at[wid])


def _k_h1(x_ref, w1_ref, o_ref):
    o_ref[...] = jnp.dot(x_ref[...], w1_ref[...],
                         preferred_element_type=jnp.float32)


def _k_u1(deg_ref, h1_ref, o_u1, o_dinv):
    deg = deg_ref[0, :, 0:1] + deg_ref[1, :, 0:1] + 1.0
    dinv = lax.rsqrt(deg)
    o_dinv[...] = dinv
    o_u1[...] = h1_ref[...] * dinv


def _k_u2(raw_ref, u1_ref, dinv_ref, b1_ref, o_u2):
    t = raw_ref[0] + raw_ref[1] + u1_ref[...]
    rel = jnp.maximum(dinv_ref[...] * t + b1_ref[...], 0.0)
    o_u2[...] = dinv_ref[...] * rel


def _k_ab(raw_ref, u2_ref, dinv_ref, w2_ref, b2_ref, l1wa_ref, l1wb_ref,
          l1b_ref, o_a, o_b):
    pre = dinv_ref[...] * (raw_ref[0] + raw_ref[1] + u2_ref[...])
    h = jnp.dot(pre, w2_ref[...], preferred_element_type=jnp.float32)
    h = h + b2_ref[...]
    o_a[...] = jnp.dot(h, l1wa_ref[...], preferred_element_type=jnp.float32)
    o_b[...] = (jnp.dot(h, l1wb_ref[...], preferred_element_type=jnp.float32)
                + l1b_ref[...])


def kernel(x, edge_index, reachability_edge_index,
           W1, b1, W2, b2, L1W, L1b, L2W, L2b):
    f32 = jnp.float32
    xp = jnp.pad(x, ((0, NPAD - N_NODES), (0, 0)))

    pad_e = E_PAD - N_EDGES
    src3 = jnp.concatenate(
        [edge_index[0], jnp.full((pad_e,), NPAD - 1, jnp.int32)]
    ).reshape(NW, NBE, G)
    dst3 = jnp.concatenate(
        [edge_index[1], jnp.full((pad_e,), NPAD - 1, jnp.int32)]
    ).reshape(NW, NBE, G)

    pad_r = R_PAD - N_REACH
    rs3 = jnp.concatenate(
        [reachability_edge_index[:, 0], jnp.zeros((pad_r,), jnp.int32)]
    ).reshape(NW, NBR, G)
    rt3 = jnp.concatenate(
        [reachability_edge_index[:, 1], jnp.zeros((pad_r,), jnp.int32)]
    ).reshape(NW, NBR, G)

    zeros_nf = jnp.zeros((NPAD, HIDDEN), f32)
    ones_rows = jnp.ones((G, HIDDEN), f32)
    l2w_flat = L2W[:, 0]
    l2b_splat = jnp.broadcast_to(L2b, (L,))

    h1 = pl.pallas_call(
        _k_h1,
        out_shape=jax.ShapeDtypeStruct((NPAD, HIDDEN), f32),
    )(xp, W1)

    deg = _k_deg(dst3, zeros_nf, ones_rows)

    u1, dinv = pl.pallas_call(
        _k_u1,
        out_shape=(jax.ShapeDtypeStruct((NPAD, HIDDEN), f32),
                   jax.ShapeDtypeStruct((NPAD, 1), f32)),
    )(deg, h1)

    raw1 = _k_scatter(u1, src3, dst3, zeros_nf)

    u2 = pl.pallas_call(
        _k_u2,
        out_shape=jax.ShapeDtypeStruct((NPAD, HIDDEN), f32),
    )(raw1, u1, dinv, b1[None, :])

    raw2 = _k_scatter(u2, src3, dst3, zeros_nf)

    A, B = pl.pallas_call(
        _k_ab,
        out_shape=(jax.ShapeDtypeStruct((NPAD, D_FEAT), f32),
                   jax.ShapeDtypeStruct((NPAD, D_FEAT), f32)),
    )(raw2, u2, dinv, W2, b2[None, :], L1W[:D_FEAT], L1W[D_FEAT:], L1b[None, :])

    out = _k_decode(A, B, rs3, rt3, l2w_flat, l2b_splat)

    return out.reshape(-1)[:N_REACH].reshape(N_REACH, 1)


# speedup vs baseline: 7.9100x; 1.8624x over previous
import functools

import jax
import jax.numpy as jnp
from jax import lax
from jax.experimental import pallas as pl
from jax.experimental.pallas import tpu as pltpu
from jax.experimental.pallas import tpu_sc as plsc

N_NODES = 10000
D_FEAT = 128
HIDDEN = 16
N_EDGES = 320000
N_REACH = 500000

NC = 2
NS = 16
NW = NC * NS
L = 16

NPAD = 10240
ROWS_PER_TILE = NPAD // NS

G = 128
KQ = 8
NBE = 80
E_TILE = NBE * G
E_PAD = NW * E_TILE
NGE = NBE // KQ

NBR = 126
R_TILE = NBR * G
R_PAD = NW * R_TILE

_sc_mesh = plsc.VectorSubcoreMesh(core_axis_name="c", subcore_axis_name="s")
_sc_params = pltpu.CompilerParams(use_tc_tiling_on_sc=False,
                                  needs_layout_passes=False)


@functools.partial(
    pl.kernel,
    out_type=jax.ShapeDtypeStruct((NC, NPAD, HIDDEN), jnp.float32),
    mesh=_sc_mesh,
    compiler_params=_sc_params,
    scratch_types=[
        pltpu.VMEM((NBE, G), jnp.int32),
        pltpu.VMEM((G, HIDDEN), jnp.float32),
        pltpu.VMEM_SHARED((NPAD, HIDDEN), jnp.float32),
        pltpu.SemaphoreType.DMA,
    ],
)
def _k_deg(dst_hbm, zeros_hbm, ones_hbm, out_hbm, idx_v, ones_v, acc_sh, sem):
    c = lax.axis_index("c")
    s = lax.axis_index("s")
    wid = c * NS + s
    sl = pl.ds(s * ROWS_PER_TILE, ROWS_PER_TILE)
    pltpu.sync_copy(zeros_hbm.at[sl], acc_sh.at[sl])
    pltpu.sync_copy(ones_hbm, ones_v)
    pltpu.sync_copy(dst_hbm.at[wid], idx_v)
    plsc.subcore_barrier()

    def body(g, carry):
        for k in range(KQ):
            pltpu.async_copy(ones_v, acc_sh.at[idx_v.at[g * KQ + k]], sem,
                             add=True)
        for k in range(KQ):
            pltpu.make_async_copy(ones_v, acc_sh.at[idx_v.at[g * KQ + k]],
                                  sem).wait()
        return carry

    lax.fori_loop(0, NGE, body, 0)
    plsc.subcore_barrier()
    pltpu.sync_copy(acc_sh.at[sl], out_hbm.at[c, sl])


@functools.partial(
    pl.kernel,
    out_type=jax.ShapeDtypeStruct((NC, NPAD, HIDDEN), jnp.float32),
    mesh=_sc_mesh,
    compiler_params=_sc_params,
    scratch_types=[
        pltpu.VMEM((NBE, G), jnp.int32),
        pltpu.VMEM((NBE, G), jnp.int32),
        pltpu.VMEM((KQ, G, HIDDEN), jnp.float32),
        pltpu.VMEM_SHARED((NPAD, HIDDEN), jnp.float32),
        pltpu.SemaphoreType.DMA,
        pltpu.SemaphoreType.DMA,
    ],
)
def _k_scatter(u_hbm, src_hbm, dst_hbm, zeros_hbm, out_hbm,
               si_v, di_v, rows_v, acc_sh, sem_g, sem_s):
    c = lax.axis_index("c")
    s = lax.axis_index("s")
    wid = c * NS + s
    sl = pl.ds(s * ROWS_PER_TILE, ROWS_PER_TILE)
    pltpu.sync_copy(zeros_hbm.at[sl], acc_sh.at[sl])
    pltpu.sync_copy(src_hbm.at[wid], si_v)
    pltpu.sync_copy(dst_hbm.at[wid], di_v)
    plsc.subcore_barrier()

    def body(g, carry):
        for k in range(KQ):
            pltpu.async_copy(u_hbm.at[si_v.at[g * KQ + k]], rows_v.at[k],
                             sem_g)
        for k in range(KQ):
            pltpu.make_async_copy(u_hbm.at[si_v.at[g * KQ + k]], rows_v.at[k],
                                  sem_g).wait()
        for k in range(KQ):
            pltpu.async_copy(rows_v.at[k], acc_sh.at[di_v.at[g * KQ + k]],
                             sem_s, add=True)
        for k in range(KQ):
            pltpu.make_async_copy(rows_v.at[k], acc_sh.at[di_v.at[g * KQ + k]],
                                  sem_s).wait()
        return carry

    lax.fori_loop(0, NGE, body, 0)
    plsc.subcore_barrier()
    pltpu.sync_copy(acc_sh.at[sl], out_hbm.at[c, sl])


@functools.partial(
    pl.kernel,
    out_type=jax.ShapeDtypeStruct((NW, R_TILE), jnp.float32),
    mesh=_sc_mesh,
    compiler_params=_sc_params,
    scratch_types=[
        pltpu.VMEM((NBR, G), jnp.int32),
        pltpu.VMEM((NBR, G), jnp.int32),
        pltpu.VMEM((G, D_FEAT), jnp.float32),
        pltpu.VMEM((G, D_FEAT), jnp.float32),
        pltpu.VMEM((G, D_FEAT), jnp.float32),
        pltpu.VMEM((D_FEAT,), jnp.float32),
        pltpu.VMEM((L,), jnp.float32),
        pltpu.VMEM((R_TILE,), jnp.float32),
        pltpu.SemaphoreType.DMA,
        pltpu.SemaphoreType.DMA,
        pltpu.SemaphoreType.DMA,
        pltpu.SemaphoreType.DMA,
        pltpu.SemaphoreType.DMA,
        pltpu.SemaphoreType.DMA,
    ],
)
def _k_decode(a_hbm, b_hbm, s_hbm, t_hbm, w_hbm, l2b_hbm, out_hbm,
              si_v, ti_v, r0, r1, r2, w_v, l2b_v, out_v,
              sa0, sa1, sa2, sb0, sb1, sb2):
    bufs = (r0, r1, r2)
    sas = (sa0, sa1, sa2)
    sbs = (sb0, sb1, sb2)
    c = lax.axis_index("c")
    s = lax.axis_index("s")
    wid = c * NS + s
    pltpu.sync_copy(s_hbm.at[wid], si_v)
    pltpu.sync_copy(t_hbm.at[wid], ti_v)
    pltpu.sync_copy(w_hbm, w_v)
    pltpu.sync_copy(l2b_hbm, l2b_v)
    bias = l2b_v[...]
    row_ids = [lax.iota(jnp.int32, L) + (L * g) for g in range(G // L)]
    zero16 = jnp.zeros((L,), jnp.float32)

    def start_a(bb, p):
        pltpu.async_copy(a_hbm.at[si_v.at[bb]], bufs[p], sas[p])

    def wait_a(p):
        pltpu.make_async_copy(a_hbm.at[si_v.at[0]], bufs[p], sas[p]).wait()

    def start_b(bb, p):
        pltpu.async_copy(b_hbm.at[ti_v.at[bb]], bufs[p], sbs[p], add=True)

    def wait_b(p):
        pltpu.make_async_copy(b_hbm.at[ti_v.at[0]], bufs[p], sbs[p]).wait()

    def compute(bb, buf):
        def jbody(j, accs):
            col = jnp.zeros((L,), jnp.int32) + j
            wj = plsc.load_gather(w_v, [col])
            out = []
            for g in range(G // L):
                v = plsc.load_gather(buf, [row_ids[g], col])
                out.append(accs[g] + jnp.maximum(v, 0.0) * wj)
            return tuple(out)

        accs = lax.fori_loop(0, D_FEAT, jbody,
                             tuple(zero16 for _ in range(G // L)))
        for g in range(G // L):
            z = accs[g] + bias
            out_v[pl.ds(bb * G + L * g, L)] = 1.0 / (1.0 + jnp.exp(-z))

    start_a(0, 0)
    start_a(1, 1)
    wait_a(0)
    start_b(0, 0)

    def outer(i, carry):
        for k in range(3):
            bb = i * 3 + k
            p0, p1, p2 = k % 3, (k + 1) % 3, (k + 2) % 3

            @pl.when(bb + 2 < NBR)
            def _():
                start_a(bb + 2, p2)

            @pl.when(bb + 1 < NBR)
            def _():
                wait_a(p1)
                start_b(bb + 1, p1)

            wait_b(p0)
            compute(bb, bufs[p0])
        return carry

    lax.fori_loop(0, NBR // 3, outer, 0)
    pltpu.sync_copy(out_v, out_hbm.at[wid])


def _k_h1(x_ref, w1_ref, o_ref):
    o_ref[...] = jnp.dot(x_ref[...], w1_ref[...],
                         preferred_element_type=jnp.float32)


def _k_u1(deg_ref, h1_ref, o_u1, o_dinv):
    deg = deg_ref[0, :, 0:1] + deg_ref[1, :, 0:1] + 1.0
    dinv = lax.rsqrt(deg)
    o_dinv[...] = dinv
    o_u1[...] = h1_ref[...] * dinv


def _k_u2(raw_ref, u1_ref, dinv_ref, b1_ref, o_u2):
    t = raw_ref[0] + raw_ref[1] + u1_ref[...]
    rel = jnp.maximum(dinv_ref[...] * t + b1_ref[...], 0.0)
    o_u2[...] = dinv_ref[...] * rel


def _k_ab(raw_ref, u2_ref, dinv_ref, w2_ref, b2_ref, l1wa_ref, l1wb_ref,
          l1b_ref, o_a, o_b):
    pre = dinv_ref[...] * (raw_ref[0] + raw_ref[1] + u2_ref[...])
    h = jnp.dot(pre, w2_ref[...], preferred_element_type=jnp.float32)
    h = h + b2_ref[...]
    o_a[...] = jnp.dot(h, l1wa_ref[...], preferred_element_type=jnp.float32)
    o_b[...] = (jnp.dot(h, l1wb_ref[...], preferred_element_type=jnp.float32)
                + l1b_ref[...])


def kernel(x, edge_index, reachability_edge_index,
           W1, b1, W2, b2, L1W, L1b, L2W, L2b):
    f32 = jnp.float32
    xp = jnp.pad(x, ((0, NPAD - N_NODES), (0, 0)))

    pad_e = E_PAD - N_EDGES
    src3 = jnp.concatenate(
        [edge_index[0], jnp.full((pad_e,), NPAD - 1, jnp.int32)]
    ).reshape(NW, NBE, G)
    dst3 = jnp.concatenate(
        [edge_index[1], jnp.full((pad_e,), NPAD - 1, jnp.int32)]
    ).reshape(NW, NBE, G)

    pad_r = R_PAD - N_REACH
    rs3 = jnp.concatenate(
        [reachability_edge_index[:, 0], jnp.zeros((pad_r,), jnp.int32)]
    ).reshape(NW, NBR, G)
    rt3 = jnp.concatenate(
        [reachability_edge_index[:, 1], jnp.zeros((pad_r,), jnp.int32)]
    ).reshape(NW, NBR, G)

    zeros_nf = jnp.zeros((NPAD, HIDDEN), f32)
    ones_rows = jnp.ones((G, HIDDEN), f32)
    l2w_flat = L2W[:, 0]
    l2b_splat = jnp.broadcast_to(L2b, (L,))

    h1 = pl.pallas_call(
        _k_h1,
        out_shape=jax.ShapeDtypeStruct((NPAD, HIDDEN), f32),
    )(xp, W1)

    deg = _k_deg(dst3, zeros_nf, ones_rows)

    u1, dinv = pl.pallas_call(
        _k_u1,
        out_shape=(jax.ShapeDtypeStruct((NPAD, HIDDEN), f32),
                   jax.ShapeDtypeStruct((NPAD, 1), f32)),
    )(deg, h1)

    raw1 = _k_scatter(u1, src3, dst3, zeros_nf)

    u2 = pl.pallas_call(
        _k_u2,
        out_shape=jax.ShapeDtypeStruct((NPAD, HIDDEN), f32),
    )(raw1, u1, dinv, b1[None, :])

    raw2 = _k_scatter(u2, src3, dst3, zeros_nf)

    A, B = pl.pallas_call(
        _k_ab,
        out_shape=(jax.ShapeDtypeStruct((NPAD, D_FEAT), f32),
                   jax.ShapeDtypeStruct((NPAD, D_FEAT), f32)),
    )(raw2, u2, dinv, W2, b2[None, :], L1W[:D_FEAT], L1W[D_FEAT:], L1b[None, :])

    out = _k_decode(A, B, rs3, rt3, l2w_flat, l2b_splat)

    return out.reshape(-1)[:N_REACH].reshape(N_REACH, 1)
